# baseline (speedup 1.0000x reference)
import functools

import jax
import jax.numpy as jnp
from jax import lax
from jax.experimental import pallas as pl
from jax.experimental.pallas import tpu as pltpu
from jax.experimental.pallas import tpu_sc as plsc

D_MODEL = 64
_SCALE = 8.0

_NC = 2
_NS = 16
_NW = _NC * _NS

_CHUNK = 128
_NBUF = 8
_DEPTH = 3
_ROWS_PER_ITER = 4


@functools.partial(jax.jit, static_argnames=("b_per_w", "n_chunks"))
def _sc_embed(x3, table, *, b_per_w, n_chunks):
    B = _NW * b_per_w
    mesh = plsc.VectorSubcoreMesh(
        core_axis_name="c", subcore_axis_name="s",
        num_cores=_NC, num_subcores=_NS,
    )

    @functools.partial(
        pl.kernel,
        out_type=jax.ShapeDtypeStruct((B, D_MODEL), jnp.float32),
        mesh=mesh,
        compiler_params=pltpu.CompilerParams(use_tc_tiling_on_sc=False),
        scratch_types=[
            pltpu.VMEM((n_chunks, _CHUNK), jnp.int32),
            [pltpu.VMEM((_CHUNK, D_MODEL), jnp.float32) for _ in range(_NBUF)],
            [pltpu.SemaphoreType.DMA for _ in range(_NBUF)],
            [pltpu.SemaphoreType.DMA for _ in range(_NBUF)],
        ],
    )
    def k(x_hbm, tab_hbm, out_hbm, idx_all, rows, gsem, ssem):
        wid = lax.axis_index("s") * _NC + lax.axis_index("c")
        base = wid * b_per_w

        pltpu.sync_copy(x_hbm.at[wid], idx_all)

        def start_gather(g, b):
            pltpu.async_copy(tab_hbm.at[idx_all.at[g]], rows[b], gsem[b])

        def wait_gather(g, b):
            pltpu.make_async_copy(
                tab_hbm.at[idx_all.at[g]], rows[b], gsem[b]).wait()

        def start_scatter(g, b):
            pltpu.async_copy(
                rows[b], out_hbm.at[pl.ds(base + g * _CHUNK, _CHUNK)], ssem[b])

        def wait_scatter(b):
            pltpu.make_async_copy(
                rows[b], out_hbm.at[pl.ds(base, _CHUNK)], ssem[b]).wait()

        def scale(b):
            def srows(i, carry):
                for dr in range(_ROWS_PER_ITER):
                    r = i * _ROWS_PER_ITER + dr
                    for c in range(D_MODEL // 16):
                        sl = pl.ds(16 * c, 16)
                        rows[b][r, sl] = rows[b][r, sl] * _SCALE
                return carry
            lax.fori_loop(0, _CHUNK // _ROWS_PER_ITER, srows, 0)

        for g0 in range(_DEPTH):
            start_gather(g0, g0)

        def outer(i, carry):
            for b in range(_NBUF):
                g = i * _NBUF + b
                wait_gather(g, b)
                scale(b)
                start_scatter(g, b)
                gn = g + _DEPTH
                bn = (b + _DEPTH) % _NBUF

                @pl.when(gn < n_chunks)
                def _prefetch():
                    @pl.when(gn >= _NBUF)
                    def _drain():
                        wait_scatter(bn)
                    start_gather(gn, bn)
            return carry

        lax.fori_loop(0, n_chunks // _NBUF, outer, 0)

        for b in range(_NBUF):
            wait_scatter(b)

    return k(x3, table)


def kernel(x, table):
    S, L = x.shape
    B = S * L
    b_per_w = B // _NW
    n_chunks = b_per_w // _CHUNK
    x3 = x.reshape(_NW, n_chunks, _CHUNK).astype(jnp.int32)
    out = _sc_embed(x3, table, b_per_w=b_per_w, n_chunks=n_chunks)
    return out.reshape(S, L, D_MODEL)
